# baseline (speedup 1.0000x reference)
import functools

import jax
import jax.numpy as jnp
from jax import lax
from jax.experimental import pallas as pl
from jax.experimental.pallas import tpu as pltpu
from jax.experimental.pallas import tpu_sc as plsc

VOCAB = 1000000
SEQ = 200
DIM = 64
BATCH = 1024

NC = 2
NS = 16
NW = NC * NS
ROWS = BATCH * SEQ
RPW = ROWS // NW
CHUNK = SEQ
NCHUNK = RPW // CHUNK

_mesh = plsc.VectorSubcoreMesh(core_axis_name="c", subcore_axis_name="s")

NTILE = (VOCAB + 127) // 128
TPW = NTILE // NW
ST = 2
SLANES = ST * 128
TAIL0 = NW * TPW * 128
NTAIL = VOCAB - TAIL0


@functools.partial(
    pl.kernel,
    out_type=jax.ShapeDtypeStruct((VOCAB, 2 * DIM), jnp.float32),
    mesh=_mesh,
    compiler_params=pltpu.CompilerParams(use_tc_tiling_on_sc=True,
                                         needs_layout_passes=False),
    scratch_types=[
        pltpu.VMEM((DIM, SLANES), jnp.float32),
        pltpu.VMEM((DIM, SLANES), jnp.float32),
        pltpu.VMEM((SLANES, 2 * DIM), jnp.float32),
        pltpu.VMEM((SLANES, 2 * DIM), jnp.float32),
        pltpu.SemaphoreType.DMA,
        pltpu.SemaphoreType.DMA,
        pltpu.SemaphoreType.DMA,
        pltpu.SemaphoreType.DMA,
    ],
)
def _relayout(tt_hbm, tail_hbm, scr_hbm, slab0, slab1, ov0, ov1, f0, f1,
              s0, s1):
    wid = lax.axis_index("s") * NC + lax.axis_index("c")
    lbase = wid * TPW * 128
    nslab = TPW // ST
    lanes = lax.iota(jnp.int32, 16)

    def start_fetch(si, slab, sem):
        pltpu.async_copy(tt_hbm.at[:, pl.ds(lbase + si * SLANES, SLANES)],
                         slab, sem)

    def wait_fetch(slab, sem):
        pltpu.make_async_copy(tt_hbm.at[:, pl.ds(0, SLANES)], slab,
                              sem).wait()

    def start_store(si, ov, sem):
        pltpu.async_copy(ov, scr_hbm.at[pl.ds(lbase + si * SLANES, SLANES)],
                         sem)

    def wait_store(ov, sem):
        pltpu.make_async_copy(ov, scr_hbm.at[pl.ds(0, SLANES)], sem).wait()

    def transpose(slab, ov):
        @plsc.parallel_loop(0, SLANES, 1, unroll=2)
        def _(i):
            col = jnp.full((16,), i, jnp.int32)
            for c in range(DIM // 16):
                v = plsc.load_gather(slab, [lanes + c * 16, col])
                ov[i, pl.ds(c * 16, 16)] = v

    def spair(g, _):
        si0 = 2 * g
        si1 = si0 + 1

        start_fetch(si0, slab0, f0)
        start_fetch(si1, slab1, f1)

        wait_fetch(slab0, f0)

        @pl.when(g > 0)
        def _():
            wait_store(ov0, s0)

        transpose(slab0, ov0)
        start_store(si0, ov0, s0)

        wait_fetch(slab1, f1)

        @pl.when(g > 0)
        def _():
            wait_store(ov1, s1)

        transpose(slab1, ov1)
        start_store(si1, ov1, s1)
        return 0

    lax.fori_loop(0, nslab // 2, spair, 0)
    wait_store(ov0, s0)
    wait_store(ov1, s1)

    @pl.when(wid == NW - 1)
    def _():
        def tail_slab(lo, n):
            pltpu.sync_copy(tt_hbm.at[:, pl.ds(lo, n)],
                            slab0.at[:, pl.ds(0, n)])

            @plsc.parallel_loop(0, n, 1, unroll=2)
            def _(i):
                col = jnp.full((16,), i, jnp.int32)
                for c in range(DIM // 16):
                    v = plsc.load_gather(slab0, [lanes + c * 16, col])
                    ov0[i, pl.ds(c * 16, 16)] = v

            pltpu.sync_copy(ov0.at[pl.ds(0, n)], scr_hbm.at[pl.ds(lo, n)])

        for t in range((NTAIL - 64) // 128):
            tail_slab(TAIL0 + t * 128, 128)
        pltpu.sync_copy(tail_hbm, scr_hbm.at[pl.ds(VOCAB - 64, 64)])


@functools.partial(
    pl.kernel,
    out_type=jax.ShapeDtypeStruct((BATCH, SEQ, 2 * DIM), jnp.float32),
    mesh=_mesh,
    compiler_params=pltpu.CompilerParams(use_tc_tiling_on_sc=False,
                                         needs_layout_passes=False),
    scratch_types=[
        pltpu.VMEM((RPW,), jnp.int32),
        pltpu.VMEM((CHUNK, 2 * DIM), jnp.float32),
        pltpu.VMEM((CHUNK, 2 * DIM), jnp.float32),
        pltpu.VMEM((SEQ, DIM), jnp.float32),
        pltpu.SemaphoreType.DMA,
        pltpu.SemaphoreType.DMA,
        pltpu.SemaphoreType.DMA,
        pltpu.SemaphoreType.DMA,
    ],
)
def _embed(tab_hbm, idx_hbm, pos_hbm, out_hbm,
           idx_v, rows0, rows1, pos_v, g0, g1, s0, s1):
    wid = lax.axis_index("s") * NC + lax.axis_index("c")
    base = wid * RPW
    bbase = wid * NCHUNK
    pltpu.sync_copy(idx_hbm.at[pl.ds(base, RPW)], idx_v)
    pltpu.sync_copy(pos_hbm, pos_v)

    def start_gather(ci, rows, sem):
        pltpu.async_copy(
            tab_hbm.at[idx_v.at[pl.ds(ci * CHUNK, CHUNK)]], rows, sem)

    def wait_gather(rows, sem):
        pltpu.make_async_copy(
            tab_hbm.at[idx_v.at[pl.ds(0, CHUNK)]], rows, sem).wait()

    def start_store(ci, rows, sem):
        pltpu.async_copy(rows, out_hbm.at[bbase + ci], sem)

    def wait_store(rows, sem):
        pltpu.make_async_copy(rows, out_hbm.at[bbase], sem).wait()

    def add_pos(rows):
        @plsc.parallel_loop(0, CHUNK, 1, unroll=4)
        def _(r):
            for c in range(DIM // 16):
                sl = pl.ds(c * 16, 16)
                rows[r, sl] = rows[r, sl] + pos_v[r, sl]

    def pair(g, _):
        ci0 = 2 * g
        ci1 = ci0 + 1

        @pl.when(g > 0)
        def _():
            wait_store(rows0, s0)

        start_gather(ci0, rows0, g0)

        @pl.when(g > 0)
        def _():
            wait_store(rows1, s1)

        start_gather(ci1, rows1, g1)

        wait_gather(rows0, g0)
        add_pos(rows0)
        start_store(ci0, rows0, s0)

        wait_gather(rows1, g1)
        add_pos(rows1)
        start_store(ci1, rows1, s1)
        return 0

    lax.fori_loop(0, NCHUNK // 2, pair, 0)
    wait_store(rows0, s0)
    wait_store(rows1, s1)


def kernel(x, token_table, pos_table):
    xf = x.reshape(-1).astype(jnp.int32)
    tail = jnp.pad(lax.slice(token_table, (VOCAB - 64, 0), (VOCAB, DIM)),
                   ((0, 0), (0, DIM)))
    tabp = _relayout(token_table.T, tail)
    out = _embed(tabp, xf, pos_table)
    return lax.slice(out, (0, 0, 0), (BATCH, SEQ, DIM))


# speedup vs baseline: 1.5711x; 1.5711x over previous
import functools

import jax
import jax.numpy as jnp
from jax import lax
from jax.experimental import pallas as pl
from jax.experimental.pallas import tpu as pltpu
from jax.experimental.pallas import tpu_sc as plsc

VOCAB = 1000000
SEQ = 200
DIM = 64
BATCH = 1024

NC = 2
NS = 16
NW = NC * NS
ROWS = BATCH * SEQ
RPW = ROWS // NW
CHUNK = SEQ
NCHUNK = RPW // CHUNK

_mesh = plsc.VectorSubcoreMesh(core_axis_name="c", subcore_axis_name="s")


@functools.partial(
    pl.kernel,
    out_type=jax.ShapeDtypeStruct((BATCH, SEQ, 2 * DIM), jnp.float32),
    mesh=_mesh,
    compiler_params=pltpu.CompilerParams(use_tc_tiling_on_sc=False,
                                         needs_layout_passes=False),
    scratch_types=[
        pltpu.VMEM((RPW,), jnp.int32),
        pltpu.VMEM((CHUNK, 2 * DIM), jnp.float32),
        pltpu.VMEM((CHUNK, 2 * DIM), jnp.float32),
        pltpu.VMEM((SEQ, DIM), jnp.float32),
        pltpu.SemaphoreType.DMA,
        pltpu.SemaphoreType.DMA,
        pltpu.SemaphoreType.DMA,
        pltpu.SemaphoreType.DMA,
    ],
)
def _embed(tab_hbm, idx_hbm, pos_hbm, out_hbm,
           idx_v, rows0, rows1, pos_v, g0, g1, s0, s1):
    wid = lax.axis_index("s") * NC + lax.axis_index("c")
    base = wid * RPW
    bbase = wid * NCHUNK
    pltpu.sync_copy(idx_hbm.at[pl.ds(base, RPW)], idx_v)
    pltpu.sync_copy(pos_hbm, pos_v)

    def start_gather(ci, rows, sem):
        pltpu.async_copy(
            tab_hbm.at[idx_v.at[pl.ds(ci * CHUNK, CHUNK)]], rows, sem)

    def wait_gather(rows, sem):
        pltpu.make_async_copy(
            tab_hbm.at[idx_v.at[pl.ds(0, CHUNK)]], rows, sem).wait()

    def start_store(ci, rows, sem):
        pltpu.async_copy(rows, out_hbm.at[bbase + ci], sem)

    def wait_store(rows, sem):
        pltpu.make_async_copy(rows, out_hbm.at[bbase], sem).wait()

    def add_pos(rows):
        @plsc.parallel_loop(0, CHUNK, 1, unroll=4)
        def _(r):
            for c in range(DIM // 16):
                sl = pl.ds(c * 16, 16)
                rows[r, sl] = rows[r, sl] + pos_v[r, sl]

    def pair(g, _):
        ci0 = 2 * g
        ci1 = ci0 + 1

        @pl.when(g > 0)
        def _():
            wait_store(rows0, s0)

        start_gather(ci0, rows0, g0)

        @pl.when(g > 0)
        def _():
            wait_store(rows1, s1)

        start_gather(ci1, rows1, g1)

        wait_gather(rows0, g0)
        add_pos(rows0)
        start_store(ci0, rows0, s0)

        wait_gather(rows1, g1)
        add_pos(rows1)
        start_store(ci1, rows1, s1)
        return 0

    lax.fori_loop(0, NCHUNK // 2, pair, 0)
    wait_store(rows0, s0)
    wait_store(rows1, s1)


def kernel(x, token_table, pos_table):
    xf = x.reshape(-1).astype(jnp.int32)
    tabp = jnp.pad(token_table, ((0, 0), (0, DIM)))
    out = _embed(tabp, xf, pos_table)
    return lax.slice(out, (0, 0, 0), (BATCH, SEQ, DIM))
